# baseline (speedup 1.0000x reference)
import functools

import numpy as np

import jax
import jax.numpy as jnp
from jax import lax
from jax.experimental import pallas as pl
from jax.experimental.pallas import tpu as pltpu
from jax.experimental.pallas import tpu_sc as plsc

HID = 768
B = 4
S = 2048
MAXPOS = 2048
EPS = 1e-12
L = 16
NSL = HID // L
NC = 2
NS = 16
NW = NC * NS
TOK = B * S
TPW = TOK // NW
CH = 32
NCH = TPW // CH
NWB = 2
NPB = 2
NQ = 4
QSL = NSL // NQ


def _scaledsin_table():
    pos = np.arange(MAXPOS, dtype=np.float32)
    half_d = HID // 2
    freq_seq = -np.arange(half_d, dtype=np.float32) / float(half_d)
    inv_freq = np.power(10000.0, freq_seq)
    sinusoid = pos[:, None] * inv_freq[None, :]
    tab = np.concatenate([np.sin(sinusoid), np.cos(sinusoid)], axis=-1)
    return jnp.asarray(tab, dtype=jnp.float32)


def _make_sc_kernel():
    mesh = plsc.VectorSubcoreMesh(core_axis_name="c", subcore_axis_name="s")

    @functools.partial(
        pl.kernel,
        mesh=mesh,
        out_type=jax.ShapeDtypeStruct((B, S, HID), jnp.float32),
        scratch_types=[
            pltpu.VMEM((TPW,), jnp.int32),
            pltpu.VMEM((TPW + L,), jnp.int32),
            pltpu.VMEM((NWB, CH, HID), jnp.float32),
            pltpu.VMEM((NPB, CH, HID), jnp.float32),
            pltpu.VMEM((2, HID), jnp.float32),
            pltpu.VMEM((L,), jnp.float32),
            pltpu.VMEM((HID,), jnp.float32),
            pltpu.VMEM((HID,), jnp.float32),
            pltpu.VMEM((CH, L), jnp.float32),
            pltpu.VMEM((CH, L), jnp.float32),
            pltpu.VMEM((CH, L), jnp.float32),
            pltpu.VMEM((CH, L), jnp.float32),
        ] + [pltpu.SemaphoreType.DMA] * (NWB + NPB + NWB),
    )
    def emb_kernel(word_hbm, ids_hbm, tts_hbm, pos_hbm, type_hbm, sin_hbm,
                   gamma_hbm, beta_hbm, out_hbm,
                   idx_v, tts_v, wbuf, pbuf, tbuf, sinv, gv, bv,
                   astat, a2stat, mstat, rstat,
                   *sems):
        semw = sems[0:NWB]
        semp = sems[NWB:NWB + NPB]
        semo = sems[NWB + NPB:]
        wid = lax.axis_index("s") * NC + lax.axis_index("c")
        base = wid * TPW
        s0 = base % S
        bb = base // S

        pltpu.sync_copy(ids_hbm.at[pl.ds(base, TPW)], idx_v)
        pltpu.sync_copy(tts_hbm.at[pl.ds(base, TPW)], tts_v.at[pl.ds(0, TPW)])
        pltpu.sync_copy(type_hbm, tbuf)
        pltpu.sync_copy(sin_hbm, sinv)
        pltpu.sync_copy(gamma_hbm, gv)
        pltpu.sync_copy(beta_hbm, bv)
        sv = sinv[...]
        iota = lax.iota(jnp.int32, L)

        def start_fetch(c):
            g = pltpu.async_copy(
                word_hbm.at[idx_v.at[pl.ds(c * CH, CH)]],
                wbuf.at[c % NWB], semw[c % NWB])
            p = pltpu.async_copy(
                pos_hbm.at[pl.ds(s0 + c * CH, CH)],
                pbuf.at[c % NPB], semp[c % NPB])
            return g, p

        fetches = {0: start_fetch(0), 1: start_fetch(1)}
        outcopies = {}

        for c in range(NCH):
            gw, gp = fetches.pop(c)
            gw.wait()
            gp.wait()

            w2 = wbuf.at[c % NWB]
            p2 = pbuf.at[c % NPB]

            def pass1(tp, _):
                t0 = tp * 2
                t1 = t0 + 1
                tta = tts_v[pl.ds(c * CH + t0, L)][0]
                ttb = tts_v[pl.ds(c * CH + t1, L)][0]

                def jblk(jb, carry):
                    a0, a1, q0, q1 = carry
                    for jj in range(8):
                        sl = pl.ds((jb * 8 + jj) * L, L)
                        ea = w2[t0, sl] + tbuf[tta, sl] + p2[t0, sl] * sv
                        eb = w2[t1, sl] + tbuf[ttb, sl] + p2[t1, sl] * sv
                        w2[t0, sl] = ea
                        w2[t1, sl] = eb
                        a0 = a0 + ea
                        q0 = q0 + ea * ea
                        a1 = a1 + eb
                        q1 = q1 + eb * eb
                    return a0, a1, q0, q1

                z = jnp.zeros((L,), jnp.float32)
                a0, a1, q0, q1 = lax.fori_loop(0, NSL // 8, jblk,
                                               (z, z, z, z))
                astat[t0] = a0
                a2stat[t0] = q0
                astat[t1] = a1
                a2stat[t1] = q1
                return 0

            lax.fori_loop(0, CH // 2, pass1, 0)

            def pass2(t, _):
                acc = astat[t]
                acc2 = a2stat[t]
                for d in (8, 4, 2, 1):
                    perm = iota ^ d
                    acc = acc + jnp.take(acc, perm, mode="wrap")
                    acc2 = acc2 + jnp.take(acc2, perm, mode="wrap")
                m16 = acc * (1.0 / HID)
                x = acc2 * (1.0 / HID) - m16 * m16 + EPS
                i = lax.bitcast_convert_type(x, jnp.int32)
                i = 0x5F3759DF - lax.shift_right_arithmetic(i, 1)
                y = lax.bitcast_convert_type(i, jnp.float32)
                y = y * (1.5 - 0.5 * x * y * y)
                y = y * (1.5 - 0.5 * x * y * y)
                r16 = y * (1.5 - 0.5 * x * y * y)
                mstat[t] = m16
                rstat[t] = r16
                return 0

            lax.fori_loop(0, CH, pass2, 0, unroll=2)

            for q in range(NQ):
                gq = [gv[pl.ds((q * QSL + k) * L, L)] for k in range(QSL)]
                bq = [bv[pl.ds((q * QSL + k) * L, L)] for k in range(QSL)]

                def pass3(t, _):
                    m16 = mstat[t]
                    r16 = rstat[t]
                    for k in range(QSL):
                        sl = pl.ds((q * QSL + k) * L, L)
                        w2[t, sl] = (w2[t, sl] - m16) * r16 * gq[k] + bq[k]
                    return 0

                lax.fori_loop(0, CH, pass3, 0)

            outcopies[c] = pltpu.async_copy(
                w2, out_hbm.at[bb, pl.ds(s0 + c * CH, CH)], semo[c % NWB])

            if c + 2 < NCH:
                outcopies.pop(c).wait()
                fetches[c + 2] = start_fetch(c + 2)

        for c in sorted(outcopies):
            outcopies[c].wait()

    return emb_kernel


_sc_kernel = _make_sc_kernel()


def kernel(input_ids, token_type_ids, word_table, type_table, ln_gamma,
           ln_beta, sin_scalar):
    ids = input_ids.reshape(TOK).astype(jnp.int32)
    tts = token_type_ids.reshape(TOK).astype(jnp.int32)
    pos = _scaledsin_table()
    sinv = jnp.broadcast_to(sin_scalar.astype(jnp.float32).reshape(()), (L,))
    return _sc_kernel(word_table.astype(jnp.float32), ids, tts, pos,
                     type_table.astype(jnp.float32), sinv,
                     ln_gamma.astype(jnp.float32), ln_beta.astype(jnp.float32))


# speedup vs baseline: 1.0059x; 1.0059x over previous
import functools

import numpy as np

import jax
import jax.numpy as jnp
from jax import lax
from jax.experimental import pallas as pl
from jax.experimental.pallas import tpu as pltpu
from jax.experimental.pallas import tpu_sc as plsc

HID = 768
B = 4
S = 2048
MAXPOS = 2048
EPS = 1e-12
L = 16
NSL = HID // L
NC = 2
NS = 16
NW = NC * NS
TOK = B * S
TPW = TOK // NW
CH = 32
NCH = TPW // CH
NWB = 2
NPB = 2
NQ = 4
QSL = NSL // NQ


def _scaledsin_table():
    pos = np.arange(MAXPOS, dtype=np.float32)
    half_d = HID // 2
    freq_seq = -np.arange(half_d, dtype=np.float32) / float(half_d)
    inv_freq = np.power(10000.0, freq_seq)
    sinusoid = pos[:, None] * inv_freq[None, :]
    tab = np.concatenate([np.sin(sinusoid), np.cos(sinusoid)], axis=-1)
    return jnp.asarray(tab, dtype=jnp.float32)


def _make_sc_kernel():
    mesh = plsc.VectorSubcoreMesh(core_axis_name="c", subcore_axis_name="s")

    @functools.partial(
        pl.kernel,
        mesh=mesh,
        out_type=jax.ShapeDtypeStruct((B, S, HID), jnp.float32),
        scratch_types=[
            pltpu.VMEM((TPW,), jnp.int32),
            pltpu.VMEM((TPW + L,), jnp.int32),
            pltpu.VMEM((NWB, CH, HID), jnp.float32),
            pltpu.VMEM((NPB, CH, HID), jnp.float32),
            pltpu.VMEM((2, HID), jnp.float32),
            pltpu.VMEM((L,), jnp.float32),
            pltpu.VMEM((HID,), jnp.float32),
            pltpu.VMEM((HID,), jnp.float32),
            pltpu.VMEM((CH, L), jnp.float32),
            pltpu.VMEM((CH, L), jnp.float32),
            pltpu.VMEM((CH, L), jnp.float32),
            pltpu.VMEM((CH, L), jnp.float32),
        ] + [pltpu.SemaphoreType.DMA] * (NWB + NPB + NWB),
    )
    def emb_kernel(word_hbm, ids_hbm, tts_hbm, pos_hbm, type_hbm, sin_hbm,
                   gamma_hbm, beta_hbm, out_hbm,
                   idx_v, tts_v, wbuf, pbuf, tbuf, sinv, gv, bv,
                   astat, a2stat, mstat, rstat,
                   *sems):
        semw = sems[0:NWB]
        semp = sems[NWB:NWB + NPB]
        semo = sems[NWB + NPB:]
        wid = lax.axis_index("s") * NC + lax.axis_index("c")
        base = wid * TPW
        s0 = base % S
        bb = base // S

        pltpu.sync_copy(ids_hbm.at[pl.ds(base, TPW)], idx_v)
        pltpu.sync_copy(tts_hbm.at[pl.ds(base, TPW)], tts_v.at[pl.ds(0, TPW)])
        pltpu.sync_copy(type_hbm, tbuf)
        pltpu.sync_copy(sin_hbm, sinv)
        pltpu.sync_copy(gamma_hbm, gv)
        pltpu.sync_copy(beta_hbm, bv)
        sv = sinv[...]
        iota = lax.iota(jnp.int32, L)

        def start_fetch(c):
            g = pltpu.async_copy(
                word_hbm.at[idx_v.at[pl.ds(c * CH, CH)]],
                wbuf.at[c % NWB], semw[c % NWB])
            p = pltpu.async_copy(
                pos_hbm.at[pl.ds(s0 + c * CH, CH)],
                pbuf.at[c % NPB], semp[c % NPB])
            return g, p

        fetches = {0: start_fetch(0), 1: start_fetch(1)}
        outcopies = {}

        for c in range(NCH):
            gw, gp = fetches.pop(c)
            gw.wait()
            gp.wait()

            w2 = wbuf.at[c % NWB]
            p2 = pbuf.at[c % NPB]

            def pass1(tp, _):
                t0 = tp * 2
                t1 = t0 + 1
                tta = tts_v[pl.ds(c * CH + t0, L)][0]
                ttb = tts_v[pl.ds(c * CH + t1, L)][0]

                def jblk(jb, carry):
                    a0, a1, q0, q1 = carry
                    for jj in range(4):
                        sl = pl.ds((jb * 4 + jj) * L, L)
                        ea = w2[t0, sl] + tbuf[tta, sl] + p2[t0, sl] * sv
                        eb = w2[t1, sl] + tbuf[ttb, sl] + p2[t1, sl] * sv
                        w2[t0, sl] = ea
                        w2[t1, sl] = eb
                        a0 = a0 + ea
                        q0 = q0 + ea * ea
                        a1 = a1 + eb
                        q1 = q1 + eb * eb
                    return a0, a1, q0, q1

                z = jnp.zeros((L,), jnp.float32)
                a0, a1, q0, q1 = lax.fori_loop(0, NSL // 4, jblk,
                                               (z, z, z, z))
                astat[t0] = a0
                a2stat[t0] = q0
                astat[t1] = a1
                a2stat[t1] = q1
                return 0

            lax.fori_loop(0, CH // 2, pass1, 0)

            def pass2(t, _):
                acc = astat[t]
                acc2 = a2stat[t]
                for d in (8, 4, 2, 1):
                    perm = iota ^ d
                    acc = acc + jnp.take(acc, perm, mode="wrap")
                    acc2 = acc2 + jnp.take(acc2, perm, mode="wrap")
                m16 = acc * (1.0 / HID)
                x = acc2 * (1.0 / HID) - m16 * m16 + EPS
                i = lax.bitcast_convert_type(x, jnp.int32)
                i = 0x5F3759DF - lax.shift_right_arithmetic(i, 1)
                y = lax.bitcast_convert_type(i, jnp.float32)
                y = y * (1.5 - 0.5 * x * y * y)
                y = y * (1.5 - 0.5 * x * y * y)
                r16 = y * (1.5 - 0.5 * x * y * y)
                mstat[t] = m16
                rstat[t] = r16
                return 0

            lax.fori_loop(0, CH, pass2, 0)

            for q in range(NQ):
                gq = [gv[pl.ds((q * QSL + k) * L, L)] for k in range(QSL)]
                bq = [bv[pl.ds((q * QSL + k) * L, L)] for k in range(QSL)]

                def pass3(t, _):
                    m16 = mstat[t]
                    r16 = rstat[t]
                    for k in range(QSL):
                        sl = pl.ds((q * QSL + k) * L, L)
                        w2[t, sl] = (w2[t, sl] - m16) * r16 * gq[k] + bq[k]
                    return 0

                lax.fori_loop(0, CH, pass3, 0)

            outcopies[c] = pltpu.async_copy(
                w2, out_hbm.at[bb, pl.ds(s0 + c * CH, CH)], semo[c % NWB])

            if c + 2 < NCH:
                outcopies.pop(c).wait()
                fetches[c + 2] = start_fetch(c + 2)

        for c in sorted(outcopies):
            outcopies[c].wait()

    return emb_kernel


_sc_kernel = _make_sc_kernel()


def kernel(input_ids, token_type_ids, word_table, type_table, ln_gamma,
           ln_beta, sin_scalar):
    ids = input_ids.reshape(TOK).astype(jnp.int32)
    tts = token_type_ids.reshape(TOK).astype(jnp.int32)
    pos = _scaledsin_table()
    sinv = jnp.broadcast_to(sin_scalar.astype(jnp.float32).reshape(()), (L,))
    return _sc_kernel(word_table.astype(jnp.float32), ids, tts, pos,
                     type_table.astype(jnp.float32), sinv,
                     ln_gamma.astype(jnp.float32), ln_beta.astype(jnp.float32))
